# baseline (speedup 1.0000x reference)
import functools

import jax
import jax.numpy as jnp
from jax import lax
from jax.experimental import pallas as pl
from jax.experimental.pallas import tpu as pltpu
from jax.experimental.pallas import tpu_sc as plsc

_NC = 2
_NS = 16
_C = 128
_N_PAD = 10240


def _sc_segment_sum(x, ei_flat, n_edges):
  n_nodes, d = x.shape
  n_pad = _N_PAD
  assert n_edges % 8 == 0
  rows_per_tile = n_pad // _NS
  assert rows_per_tile % _C == 0
  nw = _NC * _NS
  nfull = n_edges // _C
  tail = n_edges % _C
  per, rem = divmod(nfull, nw)
  assert per % 2 == 0 and per >= 4

  mesh = plsc.VectorSubcoreMesh(
      core_axis_name="c", subcore_axis_name="s",
      num_cores=_NC, num_subcores=_NS)

  @functools.partial(
      pl.kernel,
      out_type=jax.ShapeDtypeStruct((_NC, n_pad, d), jnp.float32),
      mesh=mesh,
      scratch_types=[
          pltpu.VMEM_SHARED((n_pad, d), jnp.float32),
          pltpu.VMEM((4, _C), jnp.int32),
          pltpu.VMEM((4, _C), jnp.int32),
          pltpu.VMEM((_C, d), jnp.float32),
          pltpu.VMEM((_C, d), jnp.float32),
          pltpu.SemaphoreType.DMA,
          pltpu.SemaphoreType.DMA,
          pltpu.SemaphoreType.DMA,
          pltpu.SemaphoreType.DMA,
          pltpu.SemaphoreType.DMA,
          pltpu.SemaphoreType.DMA,
      ],
  )
  def k(x_hbm, ei_hbm, out_hbm, acc, sidx, didx, rows0, rows1,
        gsem0, gsem1, isem0, isem1, isem2, isem3):
    cid = lax.axis_index("c")
    tid = lax.axis_index("s")
    w = cid * _NS + tid
    cbase = w * per + jnp.minimum(w, rem)
    rows = (rows0, rows1)
    gsem = (gsem0, gsem1)
    isem = (isem0, isem1, isem2, isem3)

    def idx_issue(j, s, guard=None):
      def go():
        off = pl.multiple_of((cbase + j) * _C, _C)
        pltpu.async_copy(ei_hbm.at[pl.ds(off, _C)], sidx.at[s], isem[s])
        pltpu.async_copy(ei_hbm.at[pl.ds(n_edges + off, _C)], didx.at[s],
                         isem[s])
      if guard is None:
        go()
      else:
        pl.when(guard)(go)

    def idx_wait(s):
      pltpu.make_async_copy(ei_hbm.at[pl.ds(0, _C)], sidx.at[s],
                            isem[s]).wait()
      pltpu.make_async_copy(ei_hbm.at[pl.ds(0, _C)], didx.at[s],
                            isem[s]).wait()

    def gather_issue(s, b):
      pltpu.async_copy(x_hbm.at[sidx.at[s]], rows[b], gsem[b])

    def gather_wait(s, b):
      pltpu.make_async_copy(x_hbm.at[sidx.at[s]], rows[b], gsem[b]).wait()

    for j in range(3):
      idx_issue(j, j)
    idx_wait(0)
    gather_issue(0, 0)

    def zrow(i, _):
      for jj in range(d // 16):
        rows1[i, pl.ds(jj * 16, 16)] = jnp.zeros((16,), jnp.float32)
      return 0
    lax.fori_loop(0, _C, zrow, 0)
    r0 = tid * rows_per_tile
    for kk in range(rows_per_tile // _C):
      pltpu.sync_copy(rows1, acc.at[pl.ds(r0 + kk * _C, _C)])
    plsc.subcore_barrier()

    def step(j, s, issue_gather, idx_guard):
      if issue_gather:
        idx_wait((s + 1) % 4)
        gather_issue((s + 1) % 4, (s + 1) % 2)
      gather_wait(s, s % 2)
      if idx_guard is not False:
        idx_issue(j + 3, (s + 3) % 4, guard=idx_guard)
      pltpu.sync_copy(rows[s % 2], acc.at[didx.at[s]], add=True)

    nq = (per - 2) // 4
    def body(q, _):
      j0 = 4 * q
      for s in range(4):
        step(j0 + s, s, True, j0 + s + 3 < per)
      return 0
    lax.fori_loop(0, nq, body, 0)
    for j in range(4 * nq, per):
      step(j, j % 4, j < per - 1, (None if j + 3 < per else False))

    if rem:
      @pl.when(w < rem)
      def _():
        off = pl.multiple_of((cbase + per) * _C, _C)
        pltpu.sync_copy(ei_hbm.at[pl.ds(off, _C)], sidx.at[0])
        pltpu.sync_copy(ei_hbm.at[pl.ds(n_edges + off, _C)], didx.at[0])
        pltpu.async_copy(x_hbm.at[sidx.at[0]], rows0, gsem0).wait()
        pltpu.sync_copy(rows0, acc.at[didx.at[0]], add=True)

    if tail:
      @pl.when(w == 0)
      def _():
        toff = nfull * _C
        pltpu.sync_copy(ei_hbm.at[pl.ds(toff, tail)],
                        sidx.at[0, pl.ds(0, tail)])
        pltpu.sync_copy(ei_hbm.at[pl.ds(n_edges + toff, tail)],
                        didx.at[0, pl.ds(0, tail)])
        pltpu.async_copy(
            x_hbm.at[sidx.at[0, pl.ds(0, tail)]], rows0.at[pl.ds(0, tail)],
            gsem0).wait()
        pltpu.sync_copy(rows0.at[pl.ds(0, tail)],
                        acc.at[didx.at[0, pl.ds(0, tail)]], add=True)
    plsc.subcore_barrier()

    pltpu.sync_copy(acc.at[pl.ds(r0, rows_per_tile)],
                    out_hbm.at[cid, pl.ds(r0, rows_per_tile)])

  return k(x, ei_flat)


def _tc_body(p_ref, x_ref, wm_ref, ws_ref, b_ref, o_ref):
  agg = p_ref[0] + p_ref[1]
  h = jnp.dot(agg, wm_ref[...], preferred_element_type=jnp.float32)
  h = h + jnp.dot(x_ref[...], ws_ref[...], preferred_element_type=jnp.float32)
  o_ref[...] = jnp.maximum(h + b_ref[...], 0.0)


def _tc_dense(partials, x, w_msg, w_self, b2):
  n, d = x.shape
  br = 2000
  grid = (n // br,)
  return pl.pallas_call(
      _tc_body,
      grid=grid,
      in_specs=[
          pl.BlockSpec((_NC, br, d), lambda i: (0, i, 0)),
          pl.BlockSpec((br, d), lambda i: (i, 0)),
          pl.BlockSpec((d, d), lambda i: (0, 0)),
          pl.BlockSpec((d, d), lambda i: (0, 0)),
          pl.BlockSpec((1, d), lambda i: (0, 0)),
      ],
      out_specs=pl.BlockSpec((br, d), lambda i: (i, 0)),
      out_shape=jax.ShapeDtypeStruct((n, d), jnp.float32),
  )(partials, x, w_msg, w_self, b2)


@jax.jit
def kernel(x, edge_index, W_msg, W_self, b):
  ei_flat = edge_index.astype(jnp.int32).reshape(-1)
  partials = _sc_segment_sum(x, ei_flat, edge_index.shape[1])
  return _tc_dense(partials, x, W_msg, W_self, b.reshape(1, -1))


# speedup vs baseline: 1.0625x; 1.0625x over previous
import functools

import jax
import jax.numpy as jnp
from jax import lax
from jax.experimental import pallas as pl
from jax.experimental.pallas import tpu as pltpu
from jax.experimental.pallas import tpu_sc as plsc

_NC = 2
_NS = 16
_C = 80
_N_PAD = 10240


def _sc_segment_sum(x, ei_flat, n_edges):
  n_nodes, d = x.shape
  n_pad = _N_PAD
  assert n_edges % 8 == 0
  rows_per_tile = n_pad // _NS
  assert rows_per_tile % _C == 0
  nw = _NC * _NS
  nfull = n_edges // _C
  tail = n_edges % _C
  per, rem = divmod(nfull, nw)
  assert per >= 8
  nq, rq = divmod(per - 2, 6)
  if rq == 0:
    nq, rq = nq - 1, 6

  mesh = plsc.VectorSubcoreMesh(
      core_axis_name="c", subcore_axis_name="s",
      num_cores=_NC, num_subcores=_NS)

  @functools.partial(
      pl.kernel,
      out_type=jax.ShapeDtypeStruct((_NC, n_pad, d), jnp.float32),
      mesh=mesh,
      scratch_types=[
          pltpu.VMEM_SHARED((n_pad, d), jnp.float32),
          pltpu.VMEM((6, _C), jnp.int32),
          pltpu.VMEM((6, _C), jnp.int32),
          pltpu.VMEM((_C, d), jnp.float32),
          pltpu.VMEM((_C, d), jnp.float32),
          pltpu.VMEM((_C, d), jnp.float32),
          pltpu.SemaphoreType.DMA,
          pltpu.SemaphoreType.DMA,
          pltpu.SemaphoreType.DMA,
          pltpu.SemaphoreType.DMA,
          pltpu.SemaphoreType.DMA,
          pltpu.SemaphoreType.DMA,
          pltpu.SemaphoreType.DMA,
          pltpu.SemaphoreType.DMA,
          pltpu.SemaphoreType.DMA,
          pltpu.SemaphoreType.DMA,
          pltpu.SemaphoreType.DMA,
          pltpu.SemaphoreType.DMA,
      ],
  )
  def k(x_hbm, ei_hbm, out_hbm, acc, sidx, didx, rows0, rows1, rows2,
        gsem0, gsem1, gsem2, ssem0, ssem1, ssem2,
        isem0, isem1, isem2, isem3, isem4, isem5):
    cid = lax.axis_index("c")
    tid = lax.axis_index("s")
    w = cid * _NS + tid
    cbase = w * per + jnp.minimum(w, rem)
    rows = (rows0, rows1, rows2)
    gsem = (gsem0, gsem1, gsem2)
    ssem = (ssem0, ssem1, ssem2)
    isem = (isem0, isem1, isem2, isem3, isem4, isem5)

    def idx_issue(j, s, guard=None):
      def go():
        off = pl.multiple_of((cbase + j) * _C, _C)
        pltpu.async_copy(ei_hbm.at[pl.ds(off, _C)], sidx.at[s], isem[s])
        pltpu.async_copy(ei_hbm.at[pl.ds(n_edges + off, _C)], didx.at[s],
                         isem[s])
      if guard is None:
        go()
      else:
        pl.when(guard)(go)

    def idx_wait(s):
      pltpu.make_async_copy(ei_hbm.at[pl.ds(0, _C)], sidx.at[s],
                            isem[s]).wait()
      pltpu.make_async_copy(ei_hbm.at[pl.ds(0, _C)], didx.at[s],
                            isem[s]).wait()

    def gather_issue(s, b):
      pltpu.async_copy(x_hbm.at[sidx.at[s]], rows[b], gsem[b])

    def gather_wait(s, b):
      pltpu.make_async_copy(x_hbm.at[sidx.at[s]], rows[b], gsem[b]).wait()

    def scat_issue(s, b):
      pltpu.async_copy(rows[b], acc.at[didx.at[s]], ssem[b], add=True)

    def scat_wait(s, b):
      pltpu.make_async_copy(rows[b], acc.at[didx.at[s]], ssem[b]).wait()

    for j in range(4):
      idx_issue(j, j)
    idx_wait(0)
    gather_issue(0, 0)

    def zrow(i, _):
      for jj in range(d // 16):
        rows1[i, pl.ds(jj * 16, 16)] = jnp.zeros((16,), jnp.float32)
      return 0
    lax.fori_loop(0, _C, zrow, 0)
    r0 = tid * rows_per_tile
    for kk in range(rows_per_tile // _C):
      pltpu.sync_copy(rows1, acc.at[pl.ds(r0 + kk * _C, _C)])
    plsc.subcore_barrier()

    def step(j, s6, b3, w_scat, w_idx_g, i_idx, guard):
      if w_scat:
        scat_wait((s6 + 4) % 6, (b3 + 1) % 3)
      if w_idx_g:
        idx_wait((s6 + 1) % 6)
        gather_issue((s6 + 1) % 6, (b3 + 1) % 3)
      if i_idx:
        idx_issue(j + 4, (s6 + 4) % 6, guard=guard)
      gather_wait(s6, b3)
      scat_issue(s6, b3)

    step(0, 0, 0, False, True, True, None)
    step(1, 1, 1, False, True, True, None)
    def body(q, _):
      j0 = 2 + 6 * q
      for t in range(6):
        j = j0 + t
        s6 = (2 + t) % 6
        b3 = (2 + t) % 3
        step(j, s6, b3, True, True, True, j + 4 < per)
      return 0
    lax.fori_loop(0, nq, body, 0)
    for j in range(per - rq, per):
      step(j, j % 6, j % 3, True, j + 1 < per, j + 4 < per, None)
    scat_wait((per - 2) % 6, (per - 2) % 3)
    scat_wait((per - 1) % 6, (per - 1) % 3)

    if rem:
      @pl.when(w < rem)
      def _():
        off = pl.multiple_of((cbase + per) * _C, _C)
        pltpu.sync_copy(ei_hbm.at[pl.ds(off, _C)], sidx.at[0])
        pltpu.sync_copy(ei_hbm.at[pl.ds(n_edges + off, _C)], didx.at[0])
        pltpu.async_copy(x_hbm.at[sidx.at[0]], rows0, gsem0).wait()
        pltpu.sync_copy(rows0, acc.at[didx.at[0]], add=True)

    if tail:
      @pl.when(w == 0)
      def _():
        toff = nfull * _C
        pltpu.sync_copy(ei_hbm.at[pl.ds(toff, tail)],
                        sidx.at[0, pl.ds(0, tail)])
        pltpu.sync_copy(ei_hbm.at[pl.ds(n_edges + toff, tail)],
                        didx.at[0, pl.ds(0, tail)])
        pltpu.async_copy(
            x_hbm.at[sidx.at[0, pl.ds(0, tail)]], rows0.at[pl.ds(0, tail)],
            gsem0).wait()
        pltpu.sync_copy(rows0.at[pl.ds(0, tail)],
                        acc.at[didx.at[0, pl.ds(0, tail)]], add=True)
    plsc.subcore_barrier()

    pltpu.sync_copy(acc.at[pl.ds(r0, rows_per_tile)],
                    out_hbm.at[cid, pl.ds(r0, rows_per_tile)])

  return k(x, ei_flat)


def _tc_body(p_ref, x_ref, wm_ref, ws_ref, b_ref, o_ref):
  agg = p_ref[0] + p_ref[1]
  h = jnp.dot(agg, wm_ref[...], preferred_element_type=jnp.float32)
  h = h + jnp.dot(x_ref[...], ws_ref[...], preferred_element_type=jnp.float32)
  o_ref[...] = jnp.maximum(h + b_ref[...], 0.0)


def _tc_dense(partials, x, w_msg, w_self, b2):
  n, d = x.shape
  br = 2000
  grid = (n // br,)
  return pl.pallas_call(
      _tc_body,
      grid=grid,
      in_specs=[
          pl.BlockSpec((_NC, br, d), lambda i: (0, i, 0)),
          pl.BlockSpec((br, d), lambda i: (i, 0)),
          pl.BlockSpec((d, d), lambda i: (0, 0)),
          pl.BlockSpec((d, d), lambda i: (0, 0)),
          pl.BlockSpec((1, d), lambda i: (0, 0)),
      ],
      out_specs=pl.BlockSpec((br, d), lambda i: (i, 0)),
      out_shape=jax.ShapeDtypeStruct((n, d), jnp.float32),
  )(partials, x, w_msg, w_self, b2)


@jax.jit
def kernel(x, edge_index, W_msg, W_self, b):
  ei_flat = edge_index.astype(jnp.int32).reshape(-1)
  partials = _sc_segment_sum(x, ei_flat, edge_index.shape[1])
  return _tc_dense(partials, x, W_msg, W_self, b.reshape(1, -1))
